# baseline (speedup 1.0000x reference)
import jax
import jax.numpy as jnp
from jax import lax
from jax.experimental import pallas as pl
from jax.experimental.pallas import tpu as pltpu
from jax.experimental.pallas import tpu_sc as plsc

N = 65536
C = 80
BG = 80
RATIO = 3

NS = 16
LAB_CH = N // NS
GR = 128
CH_RS = GR * C
NB = 8192
SL = NB // NS
LCAP = 512
BM = 4096

_INT_MIN_PY = -2**31


def _dense_body(x_ref, lab_ref, w_ref, rs_ref, scal_ref, xc_ref, sem):
    i = pl.program_id(0)
    cp = pltpu.make_async_copy(x_ref, xc_ref.at[i], sem)
    cp.start()
    xb = x_ref[...]
    labb = jnp.broadcast_to(lab_ref[...], (BM, C))
    wb = jnp.broadcast_to(w_ref[...], (BM, C))
    posb = (labb >= 0) & (labb < BG)
    ohp = (lax.broadcasted_iota(jnp.int32, (BM, C), 1) == labb) & posb
    sp = jnp.maximum(xb, 0.0) + jnp.log(1.0 + jnp.exp(-jnp.abs(xb)))
    rsel = wb * jnp.where(ohp, sp - xb, sp)
    ones = jnp.ones((C, 1), jnp.bfloat16)
    rs = lax.dot_general(rsel.astype(jnp.bfloat16), ones,
                         (((1,), (0,)), ((), ())),
                         preferred_element_type=jnp.float32)
    rs_ref[...] = rs
    np_b = jnp.sum(ohp.astype(jnp.float32))
    nn_b = jnp.sum((labb == BG).astype(jnp.float32)) * (1.0 / C)
    pc_b = jnp.sum(jnp.where(posb, rsel, 0.0))
    li = lax.broadcasted_iota(jnp.int32, (1, 128), 1)
    part = jnp.where(li == 0, np_b,
                     jnp.where(li == 1, nn_b,
                               jnp.where(li == 2, pc_b, 0.0)))

    @pl.when(i == 0)
    def _():
        scal_ref[...] = part

    @pl.when(i > 0)
    def _():
        scal_ref[...] = scal_ref[...] + part

    cp.wait()


def _dense(x, lab2, w2):
    grid = (N // BM,)
    return pl.pallas_call(
        _dense_body,
        grid=grid,
        in_specs=[
            pl.BlockSpec((BM, C), lambda i: (i, 0)),
            pl.BlockSpec((BM, 1), lambda i: (i, 0)),
            pl.BlockSpec((BM, 1), lambda i: (i, 0)),
        ],
        out_specs=[
            pl.BlockSpec((BM, 1), lambda i: (i, 0)),
            pl.BlockSpec((1, 128), lambda i: (0, 0)),
            pl.BlockSpec(memory_space=pltpu.HBM),
        ],
        out_shape=[
            jax.ShapeDtypeStruct((N, 1), jnp.float32),
            jax.ShapeDtypeStruct((1, 128), jnp.float32),
            jax.ShapeDtypeStruct((N // BM, BM, C), jnp.float32),
        ],
        scratch_shapes=[pltpu.SemaphoreType.DMA],
    )(x, lab2, w2)


def _binof(v):
    imin = jnp.int32(_INT_MIN_PY)
    bits = plsc.bitcast(v, jnp.int32)
    key = jnp.where(bits < 0, ~bits, bits | imin)
    key = key ^ imin
    return (key >> 19) + jnp.int32(NB // 2)


def _lanevec(vals, dtype):
    io = lax.iota(jnp.int32, 16)
    v = jnp.zeros((16,), dtype)
    for i, sc in enumerate(vals):
        v = jnp.where(io == i, jnp.full((16,), sc, dtype), v)
    return v


def _zero_ref(ref, nwords):
    z = jnp.zeros((16,), ref.dtype)

    def zb(i, _):
        ref[pl.ds(i * 16, 16)] = z
        return 0

    lax.fori_loop(0, nwords // 16, zb, 0, unroll=8)


def _fused_body(x_hbm, lab_hbm, rs_hbm, ascal_hbm, out_hbm, bins_hbm,
                sh_hist, sh_misc, sh_misc2, sh_part,
                lab_v, ids_v, hist_v, idx_v, xbuf, binb_v, comb_v, tmp_v,
                rs_v, list_v, all_v, pall_v, m16i_v, m16f_v, asc_v, sem):
    t = lax.axis_index("s")
    base = t * LAB_CH
    pltpu.sync_copy(lab_hbm.at[pl.ds(base, LAB_CH)], lab_v)
    _zero_ref(ids_v, LAB_CH)
    _zero_ref(hist_v, NB)

    def cscan(i, carry):
        cnt, posv = carry
        lab = lab_v[pl.ds(i * 16, 16)]
        negm = lab == BG
        posv = posv + jnp.where((lab >= 0) & (lab < BG), 1, 0)
        rowid = base + i * 16 + lax.iota(jnp.int32, 16)
        plsc.store_compressed(ids_v.at[pl.ds(cnt, 16)], rowid, mask=negm)
        cnt = cnt + plsc.all_reduce_population_count(negm)[0]
        return cnt, posv

    cnt, posv = lax.fori_loop(0, LAB_CH // 16, cscan,
                              (jnp.int32(0), jnp.zeros((16,), jnp.int32)),
                              unroll=4)
    cpos = jnp.sum(posv)

    ones_i = jnp.full((16,), 1, jnp.int32)
    nch = (cnt + GR - 1) // GR

    def hchunk(g, _):
        def cpidx(vi, _):
            idx_v[pl.ds(vi * 16, 16)] = ids_v[pl.ds(g * GR + vi * 16, 16)]
            return 0

        lax.fori_loop(0, GR // 16, cpidx, 0, unroll=8)
        pltpu.async_copy(x_hbm.at[idx_v], xbuf, sem).wait()
        nrow = jnp.minimum(GR, cnt - g * GR)

        def hrow(r, _):
            for c5 in range(5):
                v = xbuf[r, pl.ds(c5 * 16, 16)]
                b = _binof(v)
                binb_v[pl.ds(r * C + c5 * 16, 16)] = b
                plsc.addupdate_scatter(hist_v, [b], ones_i)
            return 0

        lax.fori_loop(0, nrow, hrow, 0)

        @pl.when(nch > 1)
        def _():
            pltpu.sync_copy(binb_v, bins_hbm.at[t, pl.ds(g * CH_RS, CH_RS)])
        return 0

    lax.fori_loop(0, nch, hchunk, 0)

    pltpu.sync_copy(hist_v, sh_hist.at[t])
    m16i_v[...] = _lanevec([cnt, cpos], jnp.int32)
    pltpu.sync_copy(m16i_v, sh_misc.at[pl.ds(t * 16, 16)])
    plsc.subcore_barrier()

    _zero_ref(comb_v, SL)
    for tp in range(NS):
        pltpu.sync_copy(sh_hist.at[tp, pl.ds(t * SL, SL)], tmp_v)

        def addc(i, _):
            comb_v[pl.ds(i * 16, 16)] = (comb_v[pl.ds(i * 16, 16)]
                                         + tmp_v[pl.ds(i * 16, 16)])
            return 0

        lax.fori_loop(0, SL // 16, addc, 0, unroll=8)

    pltpu.sync_copy(sh_misc, all_v)
    accv = jnp.zeros((16,), jnp.int32)
    prefv = jnp.zeros((16,), jnp.int32)
    for tp in range(NS):
        row = all_v[pl.ds(tp * 16, 16)]
        accv = accv + row
        prefv = prefv + jnp.where(t > tp, row, 0)
    m = accv[0]
    p = accv[1]
    pref = prefv[0]
    k = jnp.minimum(RATIO * p, m)

    def sumc(i, a):
        return a + comb_v[pl.ds(i * 16, 16)]

    s_slice = jnp.sum(lax.fori_loop(0, SL // 16, sumc,
                                    jnp.zeros((16,), jnp.int32), unroll=8))
    m16i_v[...] = _lanevec([s_slice], jnp.int32)
    pltpu.sync_copy(m16i_v, sh_misc2.at[pl.ds(t * 16, 16)])
    plsc.subcore_barrier()
    pltpu.sync_copy(sh_misc2, all_v)
    atv = jnp.zeros((16,), jnp.int32)
    for tp in range(NS):
        atv = atv + jnp.where(t < tp, all_v[pl.ds(tp * 16, 16)], 0)
    a_t = atv[0]

    def walk(stp, carry):
        run, bst, c1 = carry
        vi = SL // 16 - 1 - stp
        v = comb_v[pl.ds(vi * 16, 16)]
        pc = plsc.cumsum(v)
        stot = jnp.sum(v)
        above = run + stot - pc
        win = (above < k) & (k <= above + v)
        bing = t * SL + vi * 16 + lax.iota(jnp.int32, 16)
        bst = jnp.maximum(bst, jnp.max(jnp.where(win, bing, -1)))
        c1 = jnp.maximum(c1, jnp.max(jnp.where(win, above, -1)))
        return run + stot, bst, c1

    _, bst_t, c1_t = lax.fori_loop(0, SL // 16, walk,
                                   (a_t, jnp.int32(-1), jnp.int32(-1)))
    q_t = jnp.where(bst_t >= 0, k - jnp.maximum(c1_t, 0), 0)
    m16i_v[...] = _lanevec([bst_t, q_t], jnp.int32)
    pltpu.sync_copy(m16i_v, sh_misc2.at[pl.ds(t * 16, 16)])
    plsc.subcore_barrier()
    pltpu.sync_copy(sh_misc2, all_v)
    maxv = jnp.full((16,), -1, jnp.int32)
    for tp in range(NS):
        maxv = jnp.maximum(maxv, all_v[pl.ds(tp * 16, 16)])
    bst = maxv[0]
    q = jnp.maximum(maxv[1], 0)

    pltpu.sync_copy(rs_hbm.at[pl.ds(N - 16, 16)], m16f_v)
    rs_last = m16f_v[...][15]
    zf = jnp.zeros((16,), jnp.float32)

    def echunk(g, carry):
        acc, ecnt = carry

        @pl.when(nch > 1)
        def _():
            pltpu.sync_copy(bins_hbm.at[t, pl.ds(g * CH_RS, CH_RS)], binb_v)

        off = (pref + g * GR) * C
        offc = jnp.minimum(off, N - CH_RS)
        d = off - offc
        pltpu.sync_copy(rs_hbm.at[pl.ds(offc, CH_RS)],
                        rs_v.at[pl.ds(0, CH_RS)])
        nrow = jnp.minimum(GR, cnt - g * GR)

        def row(r, rc):
            racc, recnt = rc
            for c5 in range(5):
                b = binb_v[pl.ds(r * C + c5 * 16, 16)]
                s = d + r * C + c5 * 16
                sc = jnp.minimum(s, CH_RS)
                rsl = rs_v[pl.ds(sc, 16)]
                jv = off + r * C + c5 * 16 + lax.iota(jnp.int32, 16)
                rsv = jnp.where(jv < N, rsl, jnp.full((16,), rs_last))
                racc = racc + jnp.where(b > bst, rsv, 0.0)
                eq = b == bst
                ecs = jnp.minimum(recnt, LCAP - 16)
                plsc.store_compressed(list_v.at[pl.ds(ecs, 16)], rsv, mask=eq)
                recnt = recnt + plsc.all_reduce_population_count(eq)[0]
            return racc, recnt

        return lax.fori_loop(0, nrow, row, (acc, ecnt))

    acc, ecnt = lax.fori_loop(0, nch, echunk, (zf, jnp.int32(0)))
    sum_gt = jnp.sum(acc)

    m16i_v[...] = _lanevec([ecnt], jnp.int32)
    pltpu.sync_copy(m16i_v, sh_misc.at[pl.ds(t * 16, 16)])
    plsc.subcore_barrier()
    pltpu.sync_copy(sh_misc, all_v)
    ebv = jnp.zeros((16,), jnp.int32)
    for tp in range(NS):
        ebv = ebv + jnp.where(t > tp, all_v[pl.ds(tp * 16, 16)], 0)
    e_before = ebv[0]
    qt = jnp.clip(q - e_before, 0, jnp.minimum(ecnt, LCAP))

    def lsum(i, a):
        lv = list_v[pl.ds(i * 16, 16)]
        idx = i * 16 + lax.iota(jnp.int32, 16)
        return a + jnp.where(idx < qt, lv, 0.0)

    sum_eq = jnp.sum(lax.fori_loop(0, LCAP // 16, lsum, zf, unroll=4))
    part = sum_gt + sum_eq
    m16f_v[...] = _lanevec([part], jnp.float32)
    pltpu.sync_copy(m16f_v, sh_part.at[pl.ds(t * 16, 16)])
    plsc.subcore_barrier()

    @pl.when(t == 0)
    def _():
        pltpu.sync_copy(sh_part, pall_v)
        pltpu.sync_copy(ascal_hbm, asc_v)
        tot = jnp.zeros((16,), jnp.float32)
        for tp in range(NS):
            tot = tot + pall_v[pl.ds(tp * 16, 16)]
        total_neg = tot[0]
        pos_c = asc_v[pl.ds(0, 16)][2]
        count = ((k + p) * C).astype(jnp.float32)
        m16f_v[...] = _lanevec([total_neg + pos_c, count], jnp.float32)
        pltpu.sync_copy(m16f_v, out_hbm)


def _fused(x, lab, rs, ascal):
    mesh = plsc.VectorSubcoreMesh(core_axis_name="c", subcore_axis_name="s",
                                  num_cores=1, num_subcores=NS)
    fn = pl.kernel(
        _fused_body,
        out_type=(jax.ShapeDtypeStruct((16,), jnp.float32),
                  jax.ShapeDtypeStruct((NS, LAB_CH * C), jnp.int32)),
        mesh=mesh,
        compiler_params=pltpu.CompilerParams(
            needs_layout_passes=False, use_tc_tiling_on_sc=False),
        scratch_types=[
            pltpu.VMEM_SHARED((NS, NB), jnp.int32),
            pltpu.VMEM_SHARED((NS * 16,), jnp.int32),
            pltpu.VMEM_SHARED((NS * 16,), jnp.int32),
            pltpu.VMEM_SHARED((NS * 16,), jnp.float32),
            pltpu.VMEM((LAB_CH,), jnp.int32),
            pltpu.VMEM((LAB_CH,), jnp.int32),
            pltpu.VMEM((NB,), jnp.int32),
            pltpu.VMEM((GR,), jnp.int32),
            pltpu.VMEM((GR, C), jnp.float32),
            pltpu.VMEM((CH_RS,), jnp.int32),
            pltpu.VMEM((SL,), jnp.int32),
            pltpu.VMEM((SL,), jnp.int32),
            pltpu.VMEM((CH_RS + 16,), jnp.float32),
            pltpu.VMEM((LCAP,), jnp.float32),
            pltpu.VMEM((NS * 16,), jnp.int32),
            pltpu.VMEM((NS * 16,), jnp.float32),
            pltpu.VMEM((16,), jnp.int32),
            pltpu.VMEM((16,), jnp.float32),
            pltpu.VMEM((128,), jnp.float32),
            pltpu.SemaphoreType.DMA,
        ],
    )
    return fn(x, lab, rs, ascal)


def kernel(cls_score, label, weight):
    lab2 = label.reshape(N, 1)
    w2 = weight.reshape(N, 1)
    rs2, scal, xcopy = _dense(cls_score, lab2, w2)
    out, _ = _fused(xcopy.reshape(N, C), label,
                    rs2.reshape(N), scal.reshape(128))
    return out[0] / out[1]
